# baseline (speedup 1.0000x reference)
import functools

import jax
import jax.numpy as jnp
from jax import lax
from jax.experimental import pallas as pl
from jax.experimental.pallas import tpu as pltpu
from jax.experimental.pallas import tpu_sc as plsc

_LANES = 16


@functools.lru_cache(maxsize=None)
def _make_jitter_kernel(n_batch, n_chan, n_in, n_out):
    NC = 2
    NS = 16
    assert n_batch == NS
    CW = n_chan // NC
    assert CW * NC == n_chan
    CT = 128
    n_ct = CW // CT
    TW = 128
    HW = TW // 2
    HS = HW + 2
    n_chunks = n_out // TW
    assert TW * n_chunks == n_out
    assert n_in == n_out + 2
    hvec = HW // _LANES

    mesh = plsc.VectorSubcoreMesh(core_axis_name="c", subcore_axis_name="s")

    @functools.partial(
        pl.kernel,
        out_type=jax.ShapeDtypeStruct((n_batch * n_chan, n_out), jnp.float32),
        mesh=mesh,
        compiler_params=pltpu.CompilerParams(needs_layout_passes=False),
        scratch_types=[
            pltpu.VMEM((n_in,), jnp.int32),
            pltpu.VMEM((HS, n_ct, CT), jnp.float32),
            pltpu.VMEM((CW, TW), jnp.float32),
            pltpu.VMEM_SHARED((HS, n_batch, CW), jnp.float32),
            pltpu.VMEM_SHARED((HS, n_batch, CW), jnp.float32),
            pltpu.SemaphoreType.DMA,
            pltpu.SemaphoreType.DMA,
            pltpu.SemaphoreType.DMA,
        ],
    )
    def jitter(xt_hbm, idx_hbm, out_hbm, idx_v, xbuf, obuf, sbuf_a, sbuf_b,
               sem_a, sem_b, sem_o):
        b = lax.axis_index("s")
        ch_half = lax.axis_index("c")
        c0 = ch_half * CW
        row0 = b * n_chan + c0
        pltpu.sync_copy(idx_hbm.at[b], idx_v)

        def stage(ch, half):
            sbuf, sem = ((sbuf_a, sem_a), (sbuf_b, sem_b))[half]
            return (xt_hbm.at[pl.ds(ch * TW + half * HW, HS), :,
                              pl.ds(c0, CW)], sbuf, sem)

        def extract(sbuf):
            for ct in range(n_ct):
                pltpu.sync_copy(sbuf.at[:, b, pl.ds(ct * CT, CT)],
                                xbuf.at[:, ct, :])

        def gather(ch, half):
            base = ch * TW + half * HW
            ivs = [idx_v[pl.ds(base + v * _LANES + 1, _LANES)] - base
                   for v in range(hvec)]
            zsplat = jnp.zeros((_LANES,), jnp.int32)

            @plsc.parallel_loop(0, CW, unroll=4)
            def gather_body(k):
                ksplat = jnp.full((_LANES,), k, jnp.int32)
                for v in range(hvec):
                    obuf[k, pl.ds(half * HW + v * _LANES, _LANES)] = (
                        plsc.load_gather(xbuf, [ivs[v], zsplat, ksplat]))

        @pl.when(b == 0)
        def _():
            pltpu.async_copy(*stage(0, 0))

        out_h = {}
        for ch in range(n_chunks):

            @pl.when(b == 0)
            def _(ch=ch):
                pltpu.make_async_copy(*stage(ch, 0)).wait()

            plsc.subcore_barrier()

            @pl.when(b == 0)
            def _(ch=ch):
                pltpu.async_copy(*stage(ch, 1))

            extract(sbuf_a)
            if ch >= 1:
                out_h.pop(ch - 1).wait()
            gather(ch, 0)

            @pl.when(b == 0)
            def _(ch=ch):
                pltpu.make_async_copy(*stage(ch, 1)).wait()

            plsc.subcore_barrier()

            if ch + 1 < n_chunks:
                @pl.when(b == 0)
                def _(ch=ch):
                    pltpu.async_copy(*stage(ch + 1, 0))

            extract(sbuf_b)
            gather(ch, 1)

            out_h[ch] = pltpu.async_copy(
                obuf, out_hbm.at[pl.ds(row0, CW), pl.ds(ch * TW, TW)], sem_o)
        for ch in sorted(out_h):
            out_h.pop(ch).wait()

    return jitter


def kernel(x, mindex):
    B, C, T2 = x.shape
    T = T2 - 2
    idx = mindex if mindex.dtype == jnp.int32 else mindex.astype(jnp.int32)
    xt = jnp.transpose(x, (2, 0, 1))
    out = _make_jitter_kernel(B, C, T2, T)(xt, idx)
    return out.reshape(B, C, T)


# speedup vs baseline: 2.0999x; 2.0999x over previous
import functools

import jax
import jax.numpy as jnp
from jax import lax
from jax.experimental import pallas as pl
from jax.experimental.pallas import tpu as pltpu
from jax.experimental.pallas import tpu_sc as plsc

_LANES = 16


@functools.lru_cache(maxsize=None)
def _make_jitter_kernel(n_rows, n_in, n_out, rows_per_batch):
    NC = 2
    NS = 16
    NW = NC * NS
    assert n_rows % NW == 0
    rows_per_tile = n_rows // NW
    RB = 8
    assert rows_per_tile % RB == 0
    assert rows_per_batch % rows_per_tile == 0
    n_blocks = rows_per_tile // RB
    n_vec = n_out // _LANES
    assert n_out % _LANES == 0

    mesh = plsc.VectorSubcoreMesh(core_axis_name="c", subcore_axis_name="s")

    @functools.partial(
        pl.kernel,
        out_type=jax.ShapeDtypeStruct((n_rows, n_out), jnp.float32),
        mesh=mesh,
        compiler_params=pltpu.CompilerParams(needs_layout_passes=False),
        scratch_types=[
            pltpu.VMEM((n_in,), jnp.int32),
            pltpu.VMEM((RB, n_in), jnp.float32),
            pltpu.VMEM((RB, n_in), jnp.float32),
            pltpu.VMEM((RB, n_out), jnp.float32),
            pltpu.VMEM((RB, n_out), jnp.float32),
            pltpu.SemaphoreType.DMA,
            pltpu.SemaphoreType.DMA,
            pltpu.SemaphoreType.DMA,
            pltpu.SemaphoreType.DMA,
        ],
    )
    def jitter(x_hbm, idx_hbm, out_hbm, idx_v, xbuf0, xbuf1, obuf0, obuf1,
               sem_i0, sem_i1, sem_o0, sem_o1):
        wid = lax.axis_index("s") * NC + lax.axis_index("c")
        row0 = wid * rows_per_tile
        batch = row0 // rows_per_batch
        pltpu.sync_copy(idx_hbm.at[batch], idx_v)

        xbufs, obufs = (xbuf0, xbuf1), (obuf0, obuf1)
        sems_i, sems_o = (sem_i0, sem_i1), (sem_o0, sem_o1)

        def start_in(blk):
            row = row0 + blk * RB
            return pltpu.async_copy(
                x_hbm.at[pl.ds(row, RB)], xbufs[blk % 2], sems_i[blk % 2])

        def start_out(blk):
            row = row0 + blk * RB
            return pltpu.async_copy(
                obufs[blk % 2], out_hbm.at[pl.ds(row, RB)], sems_o[blk % 2])

        in_h = {0: start_in(0)}
        out_h = {}
        for blk in range(n_blocks):
            if blk + 1 < n_blocks:
                in_h[blk + 1] = start_in(blk + 1)
            in_h.pop(blk).wait()
            if blk >= 2:
                out_h.pop(blk - 2).wait()
            xbuf, obuf = xbufs[blk % 2], obufs[blk % 2]

            @plsc.parallel_loop(0, n_vec, unroll=4)
            def gather_body(ti):
                t = ti * _LANES
                iv = idx_v[pl.ds(t + 1, _LANES)]
                for r in range(RB):
                    rsplat = jnp.full((_LANES,), r, jnp.int32)
                    obuf[r, pl.ds(t, _LANES)] = plsc.load_gather(
                        xbuf, [rsplat, iv])

            out_h[blk] = start_out(blk)
        for blk in sorted(out_h):
            out_h.pop(blk).wait()

    return jitter


def kernel(x, mindex):
    B, C, T2 = x.shape
    T = T2 - 2
    idx = mindex if mindex.dtype == jnp.int32 else mindex.astype(jnp.int32)
    x2 = x.reshape(B * C, T2)
    out = _make_jitter_kernel(B * C, T2, T, C)(x2, idx)
    return out.reshape(B, C, T)
